# Initial kernel scaffold; baseline (speedup 1.0000x reference)
import jax
import jax.numpy as jnp
from jax.experimental import pallas as pl


def kernel(x, edge_index, edge_attr, batch, inter_graph_idx, be1_w1, be1_w2, mlp1_w1, mlp1_w2, eps1, be2_w1, be2_w2, mlp2_w1, mlp2_w2, eps2, fc1_w, fc1_b, fc2_w, fc2_b, fc3_w, fc3_b):
    raise NotImplementedError("write your pallas kernel here")


# speedup vs baseline: 2.4607x; 2.4607x over previous
import functools
import jax
import jax.numpy as jnp
from jax import lax
from jax.experimental import pallas as pl
from jax.experimental.pallas import tpu as pltpu
from jax.experimental.pallas import tpu_sc as plsc

F32 = jnp.float32
I32 = jnp.int32

_N = 100000
_E = 1600000
_G = 64
_M = 8
_DIM = 64

_NP = 102400
_BN = 2048

_C = 640
_KJ = _C // 128
_NCHUNK = _E // _C
_NPT = _NP // 16

_BE = 2000


def _edge_emb_body(ea_ref, w11_ref, w12_ref, w21_ref, w22_ref, e_ref):
    ea = ea_ref[...]
    t1 = jnp.maximum(jnp.dot(ea, w11_ref[...], preferred_element_type=F32, precision=lax.Precision.HIGHEST), 0.0)
    e1 = jnp.dot(t1, w12_ref[...], preferred_element_type=F32, precision=lax.Precision.HIGHEST)
    t2 = jnp.maximum(jnp.dot(ea, w21_ref[...], preferred_element_type=F32, precision=lax.Precision.HIGHEST), 0.0)
    e2 = jnp.dot(t2, w22_ref[...], preferred_element_type=F32, precision=lax.Precision.HIGHEST)
    e_ref[...] = jnp.concatenate(
        [e2, e1, jnp.zeros((ea.shape[0], 36), F32)], axis=1)


def _edge_emb(ea, w11, w12, w21, w22):
    return pl.pallas_call(
        _edge_emb_body,
        grid=(_E // _BE,),
        in_specs=[
            pl.BlockSpec((_BE, 3), lambda i: (i, 0)),
            pl.BlockSpec((3, 28), lambda i: (0, 0)),
            pl.BlockSpec((28, 28), lambda i: (0, 0)),
            pl.BlockSpec((3, _DIM), lambda i: (0, 0)),
            pl.BlockSpec((_DIM, _DIM), lambda i: (0, 0)),
        ],
        out_specs=pl.BlockSpec((_BE, 128), lambda i: (i, 0)),
        out_shape=jax.ShapeDtypeStruct((_E, 128), F32),
    )(ea, w11, w12, w21, w22)


@functools.lru_cache(maxsize=None)
def _sc_conv(S, col_base):
    SPS = S // 2
    mesh = plsc.VectorSubcoreMesh(core_axis_name="c", subcore_axis_name="s",
                                  num_cores=2, num_subcores=16)

    @functools.partial(
        pl.kernel,
        out_type=jax.ShapeDtypeStruct((S, _NP, 16), F32),
        mesh=mesh,
        scratch_types=[
            pltpu.VMEM((_KJ, 128), I32),
            pltpu.VMEM((_KJ, 128), I32),
            pltpu.VMEM((_KJ, 128), I32),
            pltpu.VMEM((_C, 16), F32),
            pltpu.VMEM((_C, 16), F32),
            pltpu.VMEM_SHARED((_NP, 16), F32),
            pltpu.SemaphoreType.DMA,
            pltpu.SemaphoreType.DMA,
        ],
        compiler_params=pltpu.CompilerParams(use_tc_tiling_on_sc=False),
    )
    def conv(xs_hbm, ep_hbm, src_hbm, dst_hbm, out_hbm,
             idxs, idx2, idxd, xrow, erow, acc, gsem, ssem):
        c = lax.axis_index("c")
        s = lax.axis_index("s")
        for qi in range(SPS):
            q = c * SPS + qi
            qN = q * _NP
            col = col_base + q * 16

            @pl.loop(0, _C, unroll=8)
            def _(r):
                erow[r, :] = jnp.zeros((16,), F32)

            for v in range(_NPT // _C):
                pltpu.sync_copy(
                    erow.at[pl.ds(0, _C)],
                    acc.at[pl.ds(s * _NPT + v * _C, _C)])
            plsc.subcore_barrier()

            @pl.loop(s, _NCHUNK, step=16)
            def _(t):
                pltpu.sync_copy(src_hbm.at[pl.ds(t * _KJ, _KJ)], idxs)
                pltpu.sync_copy(dst_hbm.at[pl.ds(t * _KJ, _KJ)], idxd)
                for j in range(_KJ):
                    for k in range(8):
                        sl = pl.ds(k * 16, 16)
                        idx2[j, sl] = idxs[j, sl] + qN
                descs = [
                    pltpu.async_copy(xs_hbm.at[idx2.at[j]],
                                     xrow.at[pl.ds(j * 128, 128)], gsem)
                    for j in range(_KJ)
                ]
                pltpu.sync_copy(
                    ep_hbm.at[pl.ds(t * _C, _C), pl.ds(col, 16)], erow)
                for d in descs:
                    d.wait()

                @plsc.parallel_loop(0, _C, unroll=8)
                def _(r):
                    xrow[r, :] = jnp.maximum(xrow[r, :] + erow[r, :], 0.0)

                sdescs = [
                    pltpu.async_copy(xrow.at[pl.ds(j * 128, 128)],
                                     acc.at[idxd.at[j]], ssem, add=True)
                    for j in range(_KJ)
                ]
                for d in sdescs:
                    d.wait()

            plsc.subcore_barrier()

            for v in range(_NPT // _C):
                pltpu.sync_copy(acc.at[pl.ds(s * _NPT + v * _C, _C)],
                                xrow.at[pl.ds(0, _C)])
                pltpu.sync_copy(
                    xrow.at[pl.ds(0, _C)],
                    out_hbm.at[q, pl.ds(s * _NPT + v * _C, _C)])
            plsc.subcore_barrier()

    return conv


def _mlp1_body(x_ref, agg_ref, w1_ref, w2_ref, eps_ref, x1_ref, x1s_ref):
    x = x_ref[...]
    agg = jnp.concatenate([agg_ref[0], agg_ref[1]], axis=1)[:, :28]
    h = (1.0 + eps_ref[0, 0]) * x + agg
    t = jnp.maximum(jnp.dot(h, w1_ref[...], preferred_element_type=F32, precision=lax.Precision.HIGHEST), 0.0)
    y = jnp.maximum(jnp.dot(t, w2_ref[...], preferred_element_type=F32, precision=lax.Precision.HIGHEST), 0.0)
    x1_ref[...] = y
    for qq in range(4):
        x1s_ref[qq] = y[:, qq * 16:(qq + 1) * 16]


def _mlp1(x, agg1, w1, w2, eps):
    return pl.pallas_call(
        _mlp1_body,
        grid=(_NP // _BN,),
        in_specs=[
            pl.BlockSpec((_BN, 28), lambda i: (i, 0)),
            pl.BlockSpec((2, _BN, 16), lambda i: (0, i, 0)),
            pl.BlockSpec((28, 28), lambda i: (0, 0)),
            pl.BlockSpec((28, _DIM), lambda i: (0, 0)),
            pl.BlockSpec((1, 1), lambda i: (0, 0)),
        ],
        out_specs=[
            pl.BlockSpec((_BN, _DIM), lambda i: (i, 0)),
            pl.BlockSpec((4, _BN, 16), lambda i: (0, i, 0)),
        ],
        out_shape=[
            jax.ShapeDtypeStruct((_NP, _DIM), F32),
            jax.ShapeDtypeStruct((4, _NP, 16), F32),
        ],
    )(x, agg1, w1, w2, eps)


def _mlp2_pool_body(x1_ref, agg_ref, b_ref, w1_ref, w2_ref, eps_ref,
                    psum_ref, cnt_ref):
    i = pl.program_id(0)

    @pl.when(i == 0)
    def _():
        psum_ref[...] = jnp.zeros_like(psum_ref)
        cnt_ref[...] = jnp.zeros_like(cnt_ref)

    x1 = x1_ref[...]
    agg = jnp.concatenate([agg_ref[qq] for qq in range(4)], axis=1)
    h = (1.0 + eps_ref[0, 0]) * x1 + agg
    t = jnp.maximum(jnp.dot(h, w1_ref[...], preferred_element_type=F32, precision=lax.Precision.HIGHEST), 0.0)
    x2 = jnp.maximum(jnp.dot(t, w2_ref[...], preferred_element_type=F32, precision=lax.Precision.HIGHEST), 0.0)
    hcat = jnp.concatenate([x1, x2], axis=1)
    b = b_ref[0, 0, :]
    oh = (lax.broadcasted_iota(I32, (_G, _BN), 0) == b[None, :]).astype(F32)
    psum_ref[...] += jnp.dot(oh, hcat, preferred_element_type=F32, precision=lax.Precision.HIGHEST)
    cnt_ref[...] += jnp.sum(oh, axis=1)[None, :]


def _mlp2_pool(x1, agg2, batch3d, w1, w2, eps):
    return pl.pallas_call(
        _mlp2_pool_body,
        grid=(_NP // _BN,),
        in_specs=[
            pl.BlockSpec((_BN, _DIM), lambda i: (i, 0)),
            pl.BlockSpec((4, _BN, 16), lambda i: (0, i, 0)),
            pl.BlockSpec((1, 1, _BN), lambda i: (i, 0, 0)),
            pl.BlockSpec((_DIM, _DIM), lambda i: (0, 0)),
            pl.BlockSpec((_DIM, _DIM), lambda i: (0, 0)),
            pl.BlockSpec((1, 1), lambda i: (0, 0)),
        ],
        out_specs=[
            pl.BlockSpec((_G, 2 * _DIM), lambda i: (0, 0)),
            pl.BlockSpec((1, _G), lambda i: (0, 0)),
        ],
        out_shape=[
            jax.ShapeDtypeStruct((_G, 2 * _DIM), F32),
            jax.ShapeDtypeStruct((1, _G), F32),
        ],
    )(x1, agg2, batch3d, w1, w2, eps)


def _tail_body(psum_ref, cnt_ref, ig_ref, w1_ref, b1_ref, w2_ref, b2_ref,
               w3_ref, b3_ref, out_ref):
    cnt = jnp.maximum(cnt_ref[0, :], 1.0)
    h = psum_ref[...] / cnt[:, None]
    h = jnp.maximum(jnp.dot(h, w1_ref[...], preferred_element_type=F32, precision=lax.Precision.HIGHEST)
                    + b1_ref[0, :][None, :], 0.0)
    h = jnp.maximum(jnp.dot(h, w2_ref[...], preferred_element_type=F32, precision=lax.Precision.HIGHEST)
                    + b2_ref[0, :][None, :], 0.0)
    ig = ig_ref[0, :]
    oh = (lax.broadcasted_iota(I32, (_M, _G), 0) == ig[None, :]).astype(F32)
    ssum = jnp.dot(oh, h, preferred_element_type=F32, precision=lax.Precision.HIGHEST)
    c2 = jnp.maximum(jnp.sum(oh, axis=1), 1.0)
    h2 = ssum / c2[:, None]
    out_ref[...] = (jnp.dot(h2, w3_ref[...], preferred_element_type=F32, precision=lax.Precision.HIGHEST)
                    + b3_ref[0, 0])


def _tail(psum, cnt, ig, w1, b1, w2, b2, w3, b3):
    return pl.pallas_call(
        _tail_body,
        out_shape=jax.ShapeDtypeStruct((_M, 1), F32),
    )(psum, cnt, ig, w1, b1, w2, b2, w3, b3)


def kernel(x, edge_index, edge_attr, batch, inter_graph_idx,
           be1_w1, be1_w2, mlp1_w1, mlp1_w2, eps1,
           be2_w1, be2_w2, mlp2_w1, mlp2_w2, eps2,
           fc1_w, fc1_b, fc2_w, fc2_b, fc3_w, fc3_b):
    src = edge_index[0].reshape(_E // 128, 128)
    dst = edge_index[1].reshape(_E // 128, 128)

    xp = jnp.pad(x, ((0, _NP - _N), (0, 4)))
    xs1 = xp.reshape(_NP, 2, 16).transpose(1, 0, 2).reshape(2 * _NP, 16)

    e_all = _edge_emb(edge_attr, be1_w1, be1_w2, be2_w1, be2_w2)
    agg1 = _sc_conv(2, 64)(xs1, e_all, src, dst)
    x_pad = jnp.pad(x, ((0, _NP - _N), (0, 0)))
    x1, x1s = _mlp1(x_pad, agg1, mlp1_w1, mlp1_w2,
                    eps1.reshape(1, 1))
    agg2 = _sc_conv(4, 0)(x1s.reshape(4 * _NP, 16), e_all, src, dst)
    batch3d = jnp.pad(batch, (0, _NP - _N),
                      constant_values=_G).reshape(_NP // _BN, 1, _BN)
    psum, cnt = _mlp2_pool(x1, agg2, batch3d,
                           mlp2_w1, mlp2_w2, eps2.reshape(1, 1))
    out = _tail(psum, cnt, inter_graph_idx.reshape(1, _G),
                fc1_w, fc1_b.reshape(1, -1),
                fc2_w, fc2_b.reshape(1, -1),
                fc3_w, fc3_b.reshape(1, 1))
    return out.reshape(-1)


# speedup vs baseline: 2.4979x; 1.0151x over previous
import functools
import jax
import jax.numpy as jnp
from jax import lax
from jax.experimental import pallas as pl
from jax.experimental.pallas import tpu as pltpu
from jax.experimental.pallas import tpu_sc as plsc

F32 = jnp.float32
I32 = jnp.int32

_N = 100000
_E = 1600000
_G = 64
_M = 8
_DIM = 64

_NP = 102400
_BN = 2048

_C = 640
_KJ = _C // 128
_NCHUNK = _E // _C
_NPT = _NP // 16

_BE = 8000


def _edge_emb_body(ea_ref, w11_ref, w12_ref, w21_ref, w22_ref, e_ref):
    ea = ea_ref[...]
    t1 = jnp.maximum(jnp.dot(ea, w11_ref[...], preferred_element_type=F32, precision=lax.Precision.HIGHEST), 0.0)
    e1 = jnp.dot(t1, w12_ref[...], preferred_element_type=F32, precision=lax.Precision.HIGHEST)
    t2 = jnp.maximum(jnp.dot(ea, w21_ref[...], preferred_element_type=F32, precision=lax.Precision.HIGHEST), 0.0)
    e2 = jnp.dot(t2, w22_ref[...], preferred_element_type=F32, precision=lax.Precision.HIGHEST)
    e_ref[...] = jnp.concatenate(
        [e2, e1, jnp.zeros((ea.shape[0], 36), F32)], axis=1)


def _edge_emb(ea, w11, w12, w21, w22):
    return pl.pallas_call(
        _edge_emb_body,
        grid=(_E // _BE,),
        in_specs=[
            pl.BlockSpec((_BE, 3), lambda i: (i, 0)),
            pl.BlockSpec((3, 28), lambda i: (0, 0)),
            pl.BlockSpec((28, 28), lambda i: (0, 0)),
            pl.BlockSpec((3, _DIM), lambda i: (0, 0)),
            pl.BlockSpec((_DIM, _DIM), lambda i: (0, 0)),
        ],
        out_specs=pl.BlockSpec((_BE, 128), lambda i: (i, 0)),
        out_shape=jax.ShapeDtypeStruct((_E, 128), F32),
    )(ea, w11, w12, w21, w22)


@functools.lru_cache(maxsize=None)
def _sc_conv(S, col_base):
    SPS = S // 2
    mesh = plsc.VectorSubcoreMesh(core_axis_name="c", subcore_axis_name="s",
                                  num_cores=2, num_subcores=16)

    @functools.partial(
        pl.kernel,
        out_type=jax.ShapeDtypeStruct((S, _NP, 16), F32),
        mesh=mesh,
        scratch_types=[
            pltpu.VMEM((_KJ, 128), I32),
            pltpu.VMEM((_KJ, 128), I32),
            pltpu.VMEM((_KJ, 128), I32),
            pltpu.VMEM((_C, 16), F32),
            pltpu.VMEM((_C, 16), F32),
            pltpu.VMEM_SHARED((_NP, 16), F32),
            pltpu.SemaphoreType.DMA,
            pltpu.SemaphoreType.DMA,
        ],
        compiler_params=pltpu.CompilerParams(use_tc_tiling_on_sc=False),
    )
    def conv(xs_hbm, ep_hbm, src_hbm, dst_hbm, out_hbm,
             idxs, idx2, idxd, xrow, erow, acc, gsem, ssem):
        c = lax.axis_index("c")
        s = lax.axis_index("s")
        for qi in range(SPS):
            q = c * SPS + qi
            qN = q * _NP
            col = col_base + q * 16

            @pl.loop(0, _C, unroll=8)
            def _(r):
                erow[r, :] = jnp.zeros((16,), F32)

            for v in range(_NPT // _C):
                pltpu.sync_copy(
                    erow.at[pl.ds(0, _C)],
                    acc.at[pl.ds(s * _NPT + v * _C, _C)])
            plsc.subcore_barrier()

            @pl.loop(s, _NCHUNK, step=16)
            def _(t):
                pltpu.sync_copy(src_hbm.at[pl.ds(t * _KJ, _KJ)], idxs)
                pltpu.sync_copy(dst_hbm.at[pl.ds(t * _KJ, _KJ)], idxd)
                for j in range(_KJ):
                    for k in range(8):
                        sl = pl.ds(k * 16, 16)
                        idx2[j, sl] = idxs[j, sl] + qN
                descs = [
                    pltpu.async_copy(xs_hbm.at[idx2.at[j]],
                                     xrow.at[pl.ds(j * 128, 128)], gsem)
                    for j in range(_KJ)
                ]
                pltpu.sync_copy(
                    ep_hbm.at[pl.ds(t * _C, _C), pl.ds(col, 16)], erow)
                for d in descs:
                    d.wait()

                @plsc.parallel_loop(0, _C, unroll=8)
                def _(r):
                    xrow[r, :] = jnp.maximum(xrow[r, :] + erow[r, :], 0.0)

                sdescs = [
                    pltpu.async_copy(xrow.at[pl.ds(j * 128, 128)],
                                     acc.at[idxd.at[j]], ssem, add=True)
                    for j in range(_KJ)
                ]
                for d in sdescs:
                    d.wait()

            plsc.subcore_barrier()

            for v in range(_NPT // _C):
                pltpu.sync_copy(acc.at[pl.ds(s * _NPT + v * _C, _C)],
                                xrow.at[pl.ds(0, _C)])
                pltpu.sync_copy(
                    xrow.at[pl.ds(0, _C)],
                    out_hbm.at[q, pl.ds(s * _NPT + v * _C, _C)])
            plsc.subcore_barrier()

    return conv


def _mlp1_body(x_ref, agg_ref, w1_ref, w2_ref, eps_ref, x1_ref, x1s_ref):
    x = x_ref[...]
    agg = jnp.concatenate([agg_ref[0], agg_ref[1]], axis=1)[:, :28]
    h = (1.0 + eps_ref[0, 0]) * x + agg
    t = jnp.maximum(jnp.dot(h, w1_ref[...], preferred_element_type=F32, precision=lax.Precision.HIGHEST), 0.0)
    y = jnp.maximum(jnp.dot(t, w2_ref[...], preferred_element_type=F32, precision=lax.Precision.HIGHEST), 0.0)
    x1_ref[...] = y
    for qq in range(4):
        x1s_ref[qq] = y[:, qq * 16:(qq + 1) * 16]


def _mlp1(x, agg1, w1, w2, eps):
    return pl.pallas_call(
        _mlp1_body,
        grid=(_NP // _BN,),
        in_specs=[
            pl.BlockSpec((_BN, 28), lambda i: (i, 0)),
            pl.BlockSpec((2, _BN, 16), lambda i: (0, i, 0)),
            pl.BlockSpec((28, 28), lambda i: (0, 0)),
            pl.BlockSpec((28, _DIM), lambda i: (0, 0)),
            pl.BlockSpec((1, 1), lambda i: (0, 0)),
        ],
        out_specs=[
            pl.BlockSpec((_BN, _DIM), lambda i: (i, 0)),
            pl.BlockSpec((4, _BN, 16), lambda i: (0, i, 0)),
        ],
        out_shape=[
            jax.ShapeDtypeStruct((_NP, _DIM), F32),
            jax.ShapeDtypeStruct((4, _NP, 16), F32),
        ],
    )(x, agg1, w1, w2, eps)


def _mlp2_pool_body(x1_ref, agg_ref, b_ref, w1_ref, w2_ref, eps_ref,
                    psum_ref, cnt_ref):
    i = pl.program_id(0)

    @pl.when(i == 0)
    def _():
        psum_ref[...] = jnp.zeros_like(psum_ref)
        cnt_ref[...] = jnp.zeros_like(cnt_ref)

    x1 = x1_ref[...]
    agg = jnp.concatenate([agg_ref[qq] for qq in range(4)], axis=1)
    h = (1.0 + eps_ref[0, 0]) * x1 + agg
    t = jnp.maximum(jnp.dot(h, w1_ref[...], preferred_element_type=F32, precision=lax.Precision.HIGHEST), 0.0)
    x2 = jnp.maximum(jnp.dot(t, w2_ref[...], preferred_element_type=F32, precision=lax.Precision.HIGHEST), 0.0)
    hcat = jnp.concatenate([x1, x2], axis=1)
    b = b_ref[0, 0, :]
    oh = (lax.broadcasted_iota(I32, (_G, _BN), 0) == b[None, :]).astype(F32)
    psum_ref[...] += jnp.dot(oh, hcat, preferred_element_type=F32, precision=lax.Precision.HIGHEST)
    cnt_ref[...] += jnp.sum(oh, axis=1)[None, :]


def _mlp2_pool(x1, agg2, batch3d, w1, w2, eps):
    return pl.pallas_call(
        _mlp2_pool_body,
        grid=(_NP // _BN,),
        in_specs=[
            pl.BlockSpec((_BN, _DIM), lambda i: (i, 0)),
            pl.BlockSpec((4, _BN, 16), lambda i: (0, i, 0)),
            pl.BlockSpec((1, 1, _BN), lambda i: (i, 0, 0)),
            pl.BlockSpec((_DIM, _DIM), lambda i: (0, 0)),
            pl.BlockSpec((_DIM, _DIM), lambda i: (0, 0)),
            pl.BlockSpec((1, 1), lambda i: (0, 0)),
        ],
        out_specs=[
            pl.BlockSpec((_G, 2 * _DIM), lambda i: (0, 0)),
            pl.BlockSpec((1, _G), lambda i: (0, 0)),
        ],
        out_shape=[
            jax.ShapeDtypeStruct((_G, 2 * _DIM), F32),
            jax.ShapeDtypeStruct((1, _G), F32),
        ],
    )(x1, agg2, batch3d, w1, w2, eps)


def _tail_body(psum_ref, cnt_ref, ig_ref, w1_ref, b1_ref, w2_ref, b2_ref,
               w3_ref, b3_ref, out_ref):
    cnt = jnp.maximum(cnt_ref[0, :], 1.0)
    h = psum_ref[...] / cnt[:, None]
    h = jnp.maximum(jnp.dot(h, w1_ref[...], preferred_element_type=F32, precision=lax.Precision.HIGHEST)
                    + b1_ref[0, :][None, :], 0.0)
    h = jnp.maximum(jnp.dot(h, w2_ref[...], preferred_element_type=F32, precision=lax.Precision.HIGHEST)
                    + b2_ref[0, :][None, :], 0.0)
    ig = ig_ref[0, :]
    oh = (lax.broadcasted_iota(I32, (_M, _G), 0) == ig[None, :]).astype(F32)
    ssum = jnp.dot(oh, h, preferred_element_type=F32, precision=lax.Precision.HIGHEST)
    c2 = jnp.maximum(jnp.sum(oh, axis=1), 1.0)
    h2 = ssum / c2[:, None]
    out_ref[...] = (jnp.dot(h2, w3_ref[...], preferred_element_type=F32, precision=lax.Precision.HIGHEST)
                    + b3_ref[0, 0])


def _tail(psum, cnt, ig, w1, b1, w2, b2, w3, b3):
    return pl.pallas_call(
        _tail_body,
        out_shape=jax.ShapeDtypeStruct((_M, 1), F32),
    )(psum, cnt, ig, w1, b1, w2, b2, w3, b3)


def kernel(x, edge_index, edge_attr, batch, inter_graph_idx,
           be1_w1, be1_w2, mlp1_w1, mlp1_w2, eps1,
           be2_w1, be2_w2, mlp2_w1, mlp2_w2, eps2,
           fc1_w, fc1_b, fc2_w, fc2_b, fc3_w, fc3_b):
    src = edge_index[0].reshape(_E // 128, 128)
    dst = edge_index[1].reshape(_E // 128, 128)

    xp = jnp.pad(x, ((0, _NP - _N), (0, 4)))
    xs1 = xp.reshape(_NP, 2, 16).transpose(1, 0, 2).reshape(2 * _NP, 16)

    e_all = _edge_emb(edge_attr, be1_w1, be1_w2, be2_w1, be2_w2)
    agg1 = _sc_conv(2, 64)(xs1, e_all, src, dst)
    x_pad = jnp.pad(x, ((0, _NP - _N), (0, 0)))
    x1, x1s = _mlp1(x_pad, agg1, mlp1_w1, mlp1_w2,
                    eps1.reshape(1, 1))
    agg2 = _sc_conv(4, 0)(x1s.reshape(4 * _NP, 16), e_all, src, dst)
    batch3d = jnp.pad(batch, (0, _NP - _N),
                      constant_values=_G).reshape(_NP // _BN, 1, _BN)
    psum, cnt = _mlp2_pool(x1, agg2, batch3d,
                           mlp2_w1, mlp2_w2, eps2.reshape(1, 1))
    out = _tail(psum, cnt, inter_graph_idx.reshape(1, _G),
                fc1_w, fc1_b.reshape(1, -1),
                fc2_w, fc2_b.reshape(1, -1),
                fc3_w, fc3_b.reshape(1, 1))
    return out.reshape(-1)


# speedup vs baseline: 4.7962x; 1.9201x over previous
import functools
import jax
import jax.numpy as jnp
from jax import lax
from jax.experimental import pallas as pl
from jax.experimental.pallas import tpu as pltpu
from jax.experimental.pallas import tpu_sc as plsc

F32 = jnp.float32
I32 = jnp.int32

_N = 100000
_E = 1600000
_G = 64
_M = 8
_DIM = 64

_NP = 102400
_BN = 2048

_C = 640
_KJ = _C // 128
_NCHUNK = _E // _C
_NPT = _NP // 16

_BE = 8000


def _edge_emb_body(ea_ref, w11_ref, w12_ref, w21_ref, w22_ref, e_ref):
    ea = ea_ref[...]
    t1 = jnp.maximum(jnp.dot(ea, w11_ref[...], preferred_element_type=F32), 0.0)
    e1 = jnp.dot(t1, w12_ref[...], preferred_element_type=F32)
    t2 = jnp.maximum(jnp.dot(ea, w21_ref[...], preferred_element_type=F32), 0.0)
    e2 = jnp.dot(t2, w22_ref[...], preferred_element_type=F32)
    e_ref[...] = jnp.concatenate(
        [e2, e1, jnp.zeros((ea.shape[0], 36), F32)], axis=1)


def _edge_emb(ea, w11, w12, w21, w22):
    return pl.pallas_call(
        _edge_emb_body,
        grid=(_E // _BE,),
        in_specs=[
            pl.BlockSpec((_BE, 3), lambda i: (i, 0)),
            pl.BlockSpec((3, 28), lambda i: (0, 0)),
            pl.BlockSpec((28, 28), lambda i: (0, 0)),
            pl.BlockSpec((3, _DIM), lambda i: (0, 0)),
            pl.BlockSpec((_DIM, _DIM), lambda i: (0, 0)),
        ],
        out_specs=pl.BlockSpec((_BE, 128), lambda i: (i, 0)),
        out_shape=jax.ShapeDtypeStruct((_E, 128), F32),
    )(ea, w11, w12, w21, w22)


@functools.lru_cache(maxsize=None)
def _sc_conv(S, col_base):
    SPS = S // 2
    mesh = plsc.VectorSubcoreMesh(core_axis_name="c", subcore_axis_name="s",
                                  num_cores=2, num_subcores=16)

    @functools.partial(
        pl.kernel,
        out_type=jax.ShapeDtypeStruct((S, _NP, 16), F32),
        mesh=mesh,
        scratch_types=[
            pltpu.VMEM((_KJ, 128), I32),
            pltpu.VMEM((_KJ, 128), I32),
            pltpu.VMEM((_KJ, 128), I32),
            pltpu.VMEM((_C, 16), F32),
            pltpu.VMEM((_C, 16), F32),
            pltpu.VMEM_SHARED((_NP, 16), F32),
            pltpu.SemaphoreType.DMA,
            pltpu.SemaphoreType.DMA,
        ],
        compiler_params=pltpu.CompilerParams(use_tc_tiling_on_sc=False),
    )
    def conv(xs_hbm, ep_hbm, src_hbm, dst_hbm, out_hbm,
             idxs, idx2, idxd, xrow, erow, acc, gsem, ssem):
        c = lax.axis_index("c")
        s = lax.axis_index("s")
        for qi in range(SPS):
            q = c * SPS + qi
            qN = q * _NP
            col = col_base + q * 16

            @pl.loop(0, _C, unroll=8)
            def _(r):
                erow[r, :] = jnp.zeros((16,), F32)

            for v in range(_NPT // _C):
                pltpu.sync_copy(
                    erow.at[pl.ds(0, _C)],
                    acc.at[pl.ds(s * _NPT + v * _C, _C)])
            plsc.subcore_barrier()

            @pl.loop(s, _NCHUNK, step=16)
            def _(t):
                pltpu.sync_copy(src_hbm.at[pl.ds(t * _KJ, _KJ)], idxs)
                pltpu.sync_copy(dst_hbm.at[pl.ds(t * _KJ, _KJ)], idxd)
                for j in range(_KJ):
                    for k in range(8):
                        sl = pl.ds(k * 16, 16)
                        idx2[j, sl] = idxs[j, sl] + qN
                descs = [
                    pltpu.async_copy(xs_hbm.at[idx2.at[j]],
                                     xrow.at[pl.ds(j * 128, 128)], gsem)
                    for j in range(_KJ)
                ]
                pltpu.sync_copy(
                    ep_hbm.at[pl.ds(t * _C, _C), pl.ds(col, 16)], erow)
                for d in descs:
                    d.wait()

                @plsc.parallel_loop(0, _C, unroll=8)
                def _(r):
                    xrow[r, :] = jnp.maximum(xrow[r, :] + erow[r, :], 0.0)

                sdescs = [
                    pltpu.async_copy(xrow.at[pl.ds(j * 128, 128)],
                                     acc.at[idxd.at[j]], ssem, add=True)
                    for j in range(_KJ)
                ]
                for d in sdescs:
                    d.wait()

            plsc.subcore_barrier()

            for v in range(_NPT // _C):
                pltpu.sync_copy(acc.at[pl.ds(s * _NPT + v * _C, _C)],
                                xrow.at[pl.ds(0, _C)])
                pltpu.sync_copy(
                    xrow.at[pl.ds(0, _C)],
                    out_hbm.at[q, pl.ds(s * _NPT + v * _C, _C)])
            plsc.subcore_barrier()

    return conv


def _mlp1_body(x_ref, agg_ref, w1_ref, w2_ref, eps_ref, x1_ref, x1s_ref):
    x = x_ref[...]
    agg = jnp.concatenate([agg_ref[0], agg_ref[1]], axis=1)[:, :28]
    h = (1.0 + eps_ref[0, 0]) * x + agg
    t = jnp.maximum(jnp.dot(h, w1_ref[...], preferred_element_type=F32), 0.0)
    y = jnp.maximum(jnp.dot(t, w2_ref[...], preferred_element_type=F32), 0.0)
    x1_ref[...] = y
    for qq in range(4):
        x1s_ref[qq] = y[:, qq * 16:(qq + 1) * 16]


def _mlp1(x, agg1, w1, w2, eps):
    return pl.pallas_call(
        _mlp1_body,
        grid=(_NP // _BN,),
        in_specs=[
            pl.BlockSpec((_BN, 28), lambda i: (i, 0)),
            pl.BlockSpec((2, _BN, 16), lambda i: (0, i, 0)),
            pl.BlockSpec((28, 28), lambda i: (0, 0)),
            pl.BlockSpec((28, _DIM), lambda i: (0, 0)),
            pl.BlockSpec((1, 1), lambda i: (0, 0)),
        ],
        out_specs=[
            pl.BlockSpec((_BN, _DIM), lambda i: (i, 0)),
            pl.BlockSpec((4, _BN, 16), lambda i: (0, i, 0)),
        ],
        out_shape=[
            jax.ShapeDtypeStruct((_NP, _DIM), F32),
            jax.ShapeDtypeStruct((4, _NP, 16), F32),
        ],
    )(x, agg1, w1, w2, eps)


def _mlp2_pool_body(x1_ref, agg_ref, b_ref, w1_ref, w2_ref, eps_ref,
                    psum_ref, cnt_ref):
    i = pl.program_id(0)

    @pl.when(i == 0)
    def _():
        psum_ref[...] = jnp.zeros_like(psum_ref)
        cnt_ref[...] = jnp.zeros_like(cnt_ref)

    x1 = x1_ref[...]
    agg = jnp.concatenate([agg_ref[qq] for qq in range(4)], axis=1)
    h = (1.0 + eps_ref[0, 0]) * x1 + agg
    t = jnp.maximum(jnp.dot(h, w1_ref[...], preferred_element_type=F32), 0.0)
    x2 = jnp.maximum(jnp.dot(t, w2_ref[...], preferred_element_type=F32), 0.0)
    hcat = jnp.concatenate([x1, x2], axis=1)
    b = b_ref[0, 0, :]
    oh = (lax.broadcasted_iota(I32, (_G, _BN), 0) == b[None, :]).astype(F32)
    psum_ref[...] += jnp.dot(oh, hcat, preferred_element_type=F32, precision=lax.Precision.HIGHEST)
    cnt_ref[...] += jnp.sum(oh, axis=1)[None, :]


def _mlp2_pool(x1, agg2, batch3d, w1, w2, eps):
    return pl.pallas_call(
        _mlp2_pool_body,
        grid=(_NP // _BN,),
        in_specs=[
            pl.BlockSpec((_BN, _DIM), lambda i: (i, 0)),
            pl.BlockSpec((4, _BN, 16), lambda i: (0, i, 0)),
            pl.BlockSpec((1, 1, _BN), lambda i: (i, 0, 0)),
            pl.BlockSpec((_DIM, _DIM), lambda i: (0, 0)),
            pl.BlockSpec((_DIM, _DIM), lambda i: (0, 0)),
            pl.BlockSpec((1, 1), lambda i: (0, 0)),
        ],
        out_specs=[
            pl.BlockSpec((_G, 2 * _DIM), lambda i: (0, 0)),
            pl.BlockSpec((1, _G), lambda i: (0, 0)),
        ],
        out_shape=[
            jax.ShapeDtypeStruct((_G, 2 * _DIM), F32),
            jax.ShapeDtypeStruct((1, _G), F32),
        ],
    )(x1, agg2, batch3d, w1, w2, eps)


def _tail_body(psum_ref, cnt_ref, ig_ref, w1_ref, b1_ref, w2_ref, b2_ref,
               w3_ref, b3_ref, out_ref):
    cnt = jnp.maximum(cnt_ref[0, :], 1.0)
    h = psum_ref[...] / cnt[:, None]
    h = jnp.maximum(jnp.dot(h, w1_ref[...], preferred_element_type=F32)
                    + b1_ref[0, :][None, :], 0.0)
    h = jnp.maximum(jnp.dot(h, w2_ref[...], preferred_element_type=F32)
                    + b2_ref[0, :][None, :], 0.0)
    ig = ig_ref[0, :]
    oh = (lax.broadcasted_iota(I32, (_M, _G), 0) == ig[None, :]).astype(F32)
    ssum = jnp.dot(oh, h, preferred_element_type=F32, precision=lax.Precision.HIGHEST)
    c2 = jnp.maximum(jnp.sum(oh, axis=1), 1.0)
    h2 = ssum / c2[:, None]
    out_ref[...] = (jnp.dot(h2, w3_ref[...], preferred_element_type=F32)
                    + b3_ref[0, 0])


def _tail(psum, cnt, ig, w1, b1, w2, b2, w3, b3):
    return pl.pallas_call(
        _tail_body,
        out_shape=jax.ShapeDtypeStruct((_M, 1), F32),
    )(psum, cnt, ig, w1, b1, w2, b2, w3, b3)


def kernel(x, edge_index, edge_attr, batch, inter_graph_idx,
           be1_w1, be1_w2, mlp1_w1, mlp1_w2, eps1,
           be2_w1, be2_w2, mlp2_w1, mlp2_w2, eps2,
           fc1_w, fc1_b, fc2_w, fc2_b, fc3_w, fc3_b):
    src = edge_index[0].reshape(_E // 128, 128)
    dst = edge_index[1].reshape(_E // 128, 128)

    xp = jnp.pad(x, ((0, _NP - _N), (0, 4)))
    xs1 = xp.reshape(_NP, 2, 16).transpose(1, 0, 2).reshape(2 * _NP, 16)

    e_all = _edge_emb(edge_attr, be1_w1, be1_w2, be2_w1, be2_w2)
    agg1 = _sc_conv(2, 64)(xs1, e_all, src, dst)
    x_pad = jnp.pad(x, ((0, _NP - _N), (0, 0)))
    x1, x1s = _mlp1(x_pad, agg1, mlp1_w1, mlp1_w2,
                    eps1.reshape(1, 1))
    agg2 = _sc_conv(4, 0)(x1s.reshape(4 * _NP, 16), e_all, src, dst)
    batch3d = jnp.pad(batch, (0, _NP - _N),
                      constant_values=_G).reshape(_NP // _BN, 1, _BN)
    psum, cnt = _mlp2_pool(x1, agg2, batch3d,
                           mlp2_w1, mlp2_w2, eps2.reshape(1, 1))
    out = _tail(psum, cnt, inter_graph_idx.reshape(1, _G),
                fc1_w, fc1_b.reshape(1, -1),
                fc2_w, fc2_b.reshape(1, -1),
                fc3_w, fc3_b.reshape(1, 1))
    return out.reshape(-1)
